# baseline (speedup 1.0000x reference)
import functools

import jax
import jax.numpy as jnp
from jax import lax
from jax.experimental import pallas as pl
from jax.experimental.pallas import tpu as pltpu
from jax.experimental.pallas import tpu_sc as plsc

_N_BUF = 16
_CHUNK_ROWS = 256
_TC_ROWS = 5120
_NC, _NS = 2, 16
_NW = _NC * _NS
_SC_CHUNK = 16


def _dma_pipeline(in_ref, out_ref, bufs, in_sems, out_sems):
    n_chunks = out_ref.shape[0] // _CHUNK_ROWS

    def copy_in(i):
        return pltpu.make_async_copy(
            in_ref.at[pl.ds(i * _CHUNK_ROWS, _CHUNK_ROWS)],
            bufs.at[i % _N_BUF],
            in_sems.at[i % _N_BUF],
        )

    def copy_out(i):
        return pltpu.make_async_copy(
            bufs.at[i % _N_BUF],
            out_ref.at[pl.ds(i * _CHUNK_ROWS, _CHUNK_ROWS)],
            out_sems.at[i % _N_BUF],
        )

    for i in range(min(_N_BUF, n_chunks)):
        copy_in(i).start()
    for i in range(n_chunks):
        copy_in(i).wait()
        copy_out(i).start()
        j = i + _N_BUF
        if j < n_chunks:
            copy_out(j - _N_BUF).wait()
            copy_in(j).start()
    for i in range(max(0, n_chunks - _N_BUF), n_chunks):
        copy_out(i).wait()


def _sc_copy(row_base, sc_rows, in_hbm, out_hbm, bufs, in_sems, out_sems):
    rows_w = sc_rows // _NW
    n_chunks = rows_w // _SC_CHUNK
    wid = lax.axis_index("s") * _NC + lax.axis_index("c")
    in_base = row_base + wid * rows_w
    out_base = wid * rows_w

    def copy_in(i):
        return pltpu.make_async_copy(
            in_hbm.at[pl.ds(in_base + i * _SC_CHUNK, _SC_CHUNK)],
            bufs.at[i % 2],
            in_sems.at[i % 2],
        )

    def copy_out(i):
        return pltpu.make_async_copy(
            bufs.at[i % 2],
            out_hbm.at[pl.ds(out_base + i * _SC_CHUNK, _SC_CHUNK)],
            out_sems.at[i % 2],
        )

    copy_in(0).start()
    copy_in(1).start()
    for i in range(n_chunks):
        copy_in(i).wait()
        copy_out(i).start()
        j = i + 2
        if j < n_chunks:
            copy_out(j - 2).wait()
            copy_in(j).start()
    copy_out(n_chunks - 2).wait()
    copy_out(n_chunks - 1).wait()


def kernel(hidden_states):
    B, S, C = hidden_states.shape
    rows = B * S
    sc_rows = rows - _TC_ROWS
    x = hidden_states.reshape(rows, C)

    out_tc = pl.pallas_call(
        _dma_pipeline,
        in_specs=[pl.BlockSpec(memory_space=pl.ANY)],
        out_specs=pl.BlockSpec(memory_space=pl.ANY),
        out_shape=jax.ShapeDtypeStruct((_TC_ROWS, C), hidden_states.dtype),
        scratch_shapes=[
            pltpu.VMEM((_N_BUF, _CHUNK_ROWS, C), hidden_states.dtype),
            pltpu.SemaphoreType.DMA((_N_BUF,)),
            pltpu.SemaphoreType.DMA((_N_BUF,)),
        ],
    )(x)

    mesh = plsc.VectorSubcoreMesh(core_axis_name="c", subcore_axis_name="s")
    out_sc = pl.kernel(
        functools.partial(_sc_copy, _TC_ROWS, sc_rows),
        mesh=mesh,
        out_type=jax.ShapeDtypeStruct((sc_rows, C), hidden_states.dtype),
        scratch_types=[
            pltpu.VMEM((2, _SC_CHUNK, C), hidden_states.dtype),
            pltpu.SemaphoreType.DMA((2,)),
            pltpu.SemaphoreType.DMA((2,)),
        ],
    )(x)

    out = jnp.concatenate([out_tc, out_sc], axis=0)
    return out.reshape(B, S, C)


# speedup vs baseline: 2.4222x; 2.4222x over previous
import jax
import jax.numpy as jnp
from jax.experimental import pallas as pl
from jax.experimental.pallas import tpu as pltpu

_N_BUF = 15
_CHUNK_ROWS = 512


def _dma_pipeline(in_ref, out_ref, bufs, in_sems, out_sems):
    n_chunks = out_ref.shape[0] // _CHUNK_ROWS

    def copy_in(i):
        return pltpu.make_async_copy(
            in_ref.at[pl.ds(i * _CHUNK_ROWS, _CHUNK_ROWS)],
            bufs.at[i % _N_BUF],
            in_sems.at[i % _N_BUF],
        )

    def copy_out(i):
        return pltpu.make_async_copy(
            bufs.at[i % _N_BUF],
            out_ref.at[pl.ds(i * _CHUNK_ROWS, _CHUNK_ROWS)],
            out_sems.at[i % _N_BUF],
        )

    for i in range(min(_N_BUF, n_chunks)):
        copy_in(i).start()
    for i in range(n_chunks):
        copy_in(i).wait()
        copy_out(i).start()
        j = i + _N_BUF
        if j < n_chunks:
            copy_out(j - _N_BUF).wait()
            copy_in(j).start()
    for i in range(max(0, n_chunks - _N_BUF), n_chunks):
        copy_out(i).wait()


def kernel(hidden_states):
    B, S, C = hidden_states.shape
    rows = B * S
    x = hidden_states.reshape(rows, C)
    out = pl.pallas_call(
        _dma_pipeline,
        in_specs=[pl.BlockSpec(memory_space=pl.ANY)],
        out_specs=pl.BlockSpec(memory_space=pl.ANY),
        out_shape=jax.ShapeDtypeStruct((rows, C), hidden_states.dtype),
        scratch_shapes=[
            pltpu.VMEM((_N_BUF, _CHUNK_ROWS, C), hidden_states.dtype),
            pltpu.SemaphoreType.DMA((_N_BUF,)),
            pltpu.SemaphoreType.DMA((_N_BUF,)),
        ],
        compiler_params=pltpu.CompilerParams(
            vmem_limit_bytes=100 * 1024 * 1024,
        ),
    )(x)
    return out.reshape(B, S, C)


# speedup vs baseline: 2.4741x; 1.0214x over previous
import jax
import jax.numpy as jnp
from jax.experimental import pallas as pl
from jax.experimental.pallas import tpu as pltpu

_N_BUF = 16
_CHUNK_ROWS = 256


def _dma_pipeline(in_ref, out_ref, bufs, in_sems, out_sems):
    n_chunks = out_ref.shape[0] // _CHUNK_ROWS

    def copy_in(i):
        return pltpu.make_async_copy(
            in_ref.at[pl.ds(i * _CHUNK_ROWS, _CHUNK_ROWS)],
            bufs.at[i % _N_BUF],
            in_sems.at[i % _N_BUF],
        )

    def copy_out(i):
        return pltpu.make_async_copy(
            bufs.at[i % _N_BUF],
            out_ref.at[pl.ds(i * _CHUNK_ROWS, _CHUNK_ROWS)],
            out_sems.at[i % _N_BUF],
        )

    for i in range(min(_N_BUF, n_chunks)):
        copy_in(i).start()
    for i in range(n_chunks):
        copy_in(i).wait()
        copy_out(i).start()
        j = i + _N_BUF
        if j < n_chunks:
            copy_out(j - _N_BUF).wait()
            copy_in(j).start()
    for i in range(max(0, n_chunks - _N_BUF), n_chunks):
        copy_out(i).wait()


def kernel(hidden_states):
    B, S, C = hidden_states.shape
    rows = B * S
    x = hidden_states.reshape(rows, C)
    out = pl.pallas_call(
        _dma_pipeline,
        in_specs=[pl.BlockSpec(memory_space=pl.ANY)],
        out_specs=pl.BlockSpec(memory_space=pl.ANY),
        out_shape=jax.ShapeDtypeStruct((rows, C), hidden_states.dtype),
        scratch_shapes=[
            pltpu.VMEM((_N_BUF, _CHUNK_ROWS, C), hidden_states.dtype),
            pltpu.SemaphoreType.DMA((_N_BUF,)),
            pltpu.SemaphoreType.DMA((_N_BUF,)),
        ],
    )(x)
    return out.reshape(B, S, C)
